# baseline (speedup 1.0000x reference)
import functools

import jax
import jax.numpy as jnp
from jax import lax
from jax.experimental import pallas as pl
from jax.experimental.pallas import tpu as pltpu
from jax.experimental.pallas import tpu_sc as plsc

WIDTH = 4096
D = 2048
BATCH = 2
ROWS = BATCH * WIDTH
NC, NS = 2, 16
NW = NC * NS
RPW = ROWS // NW
CHG = 8
CHW = 16
GPW = CHW // CHG
NG = RPW // CHG
NWR = RPW // CHW
NBUF = 3
GA = 4


def _permute_body(perm_hbm, x_hbm, out_hbm, idx_v, *rest):
    bufs = list(rest[:NBUF])
    sgs = list(rest[NBUF:NBUF + NBUF * GPW])
    sws = list(rest[NBUF + NBUF * GPW:])

    c = lax.axis_index("c")
    s = lax.axis_index("s")
    wid = s * NC + c
    batch = wid // (NW // BATCH)
    pidx = wid % (NW // BATCH)
    pbase = pidx * RPW
    out_base = batch * WIDTH + pbase
    off = batch * WIDTH

    pltpu.sync_copy(perm_hbm.at[pl.ds(pbase, RPW)], idx_v)
    for i in range(RPW // 16):
        idx_v[pl.ds(i * 16, 16)] = idx_v[pl.ds(i * 16, 16)] + off

    gathers = [None] * NG
    writes = [None] * NWR
    waited = [False] * NWR

    def g_start(g):
        wb = (g // GPW) % NBUF
        half = g % GPW
        gathers[g] = pltpu.async_copy(
            x_hbm.at[idx_v.at[pl.ds(g * CHG, CHG)]],
            bufs[wb].at[pl.ds(half * CHG, CHG)],
            sgs[wb * GPW + half])

    def w_start(w):
        wb = w % NBUF
        writes[w] = pltpu.async_copy(
            bufs[wb], out_hbm.at[pl.ds(out_base + w * CHW, CHW)], sws[wb])

    for g in range(min(GA, NG)):
        g_start(g)
    for g in range(NG):
        if g + GA < NG:
            wprev = (g + GA) // GPW - NBUF
            if wprev >= 0 and not waited[wprev]:
                writes[wprev].wait()
                waited[wprev] = True
            g_start(g + GA)
        gathers[g].wait()
        if g % GPW == GPW - 1:
            w_start(g // GPW)
    for w in range(NWR):
        if not waited[w]:
            writes[w].wait()
            waited[w] = True


@jax.jit
def _permute_flat(perm, xf):
    mesh = plsc.VectorSubcoreMesh(
        core_axis_name="c", subcore_axis_name="s",
        num_cores=NC, num_subcores=NS)
    run = pl.kernel(
        _permute_body,
        out_type=jax.ShapeDtypeStruct((ROWS, D), jnp.float32),
        mesh=mesh,
        scratch_types=(
            [pltpu.VMEM((RPW,), jnp.int32)]
            + [pltpu.VMEM((CHW, D), jnp.float32) for _ in range(NBUF)]
            + [pltpu.SemaphoreType.DMA for _ in range(NBUF * GPW)]
            + [pltpu.SemaphoreType.DMA for _ in range(NBUF)]
        ),
        name="sc_row_permute",
    )
    return run(perm, xf)


def kernel(X, perm):
    xf = X.reshape(ROWS, D)
    out = _permute_flat(perm, xf)
    return out.reshape(X.shape)


# speedup vs baseline: 1.0002x; 1.0002x over previous
import functools

import jax
import jax.numpy as jnp
from jax import lax
from jax.experimental import pallas as pl
from jax.experimental.pallas import tpu as pltpu
from jax.experimental.pallas import tpu_sc as plsc

WIDTH = 4096
D = 2048
BATCH = 2
ROWS = BATCH * WIDTH
NC, NS = 2, 16
NW = NC * NS
RPW = ROWS // NW
CHG = 8
CHW = 16
GPW = CHW // CHG
NG = RPW // CHG
NWR = RPW // CHW
NBUF = 3
GA = 4


def _permute_body(perm_hbm, x_hbm, out_hbm, idx_v, bufs_v, sg_arr, sw_arr):
    bufs = [bufs_v.at[i] for i in range(NBUF)]
    sgs = [sg_arr.at[i] for i in range(NBUF * GPW)]
    sws = [sw_arr.at[i] for i in range(NBUF)]

    c = lax.axis_index("c")
    s = lax.axis_index("s")
    wid = s * NC + c
    batch = wid // (NW // BATCH)
    pidx = wid % (NW // BATCH)
    pbase = pidx * RPW
    out_base = batch * WIDTH + pbase
    off = batch * WIDTH

    pltpu.sync_copy(perm_hbm.at[pl.ds(pbase, RPW)], idx_v)
    for i in range(RPW // 16):
        idx_v[pl.ds(i * 16, 16)] = idx_v[pl.ds(i * 16, 16)] + off

    gathers = [None] * NG
    writes = [None] * NWR
    waited = [False] * NWR

    def g_start(g):
        wb = (g // GPW) % NBUF
        half = g % GPW
        gathers[g] = pltpu.async_copy(
            x_hbm.at[idx_v.at[pl.ds(g * CHG, CHG)]],
            bufs[wb].at[pl.ds(half * CHG, CHG)],
            sgs[wb * GPW + half])

    def w_start(w):
        wb = w % NBUF
        writes[w] = pltpu.async_copy(
            bufs[wb], out_hbm.at[pl.ds(out_base + w * CHW, CHW)], sws[wb])

    for g in range(min(GA, NG)):
        g_start(g)
    for g in range(NG):
        if g + GA < NG:
            wprev = (g + GA) // GPW - NBUF
            if wprev >= 0 and not waited[wprev]:
                writes[wprev].wait()
                waited[wprev] = True
            g_start(g + GA)
        gathers[g].wait()
        if g % GPW == GPW - 1:
            w_start(g // GPW)
    for w in range(NWR):
        if not waited[w]:
            writes[w].wait()
            waited[w] = True


@jax.jit
def _permute_flat(perm, xf):
    mesh = plsc.VectorSubcoreMesh(
        core_axis_name="c", subcore_axis_name="s",
        num_cores=NC, num_subcores=NS)
    run = pl.kernel(
        _permute_body,
        out_type=jax.ShapeDtypeStruct((ROWS, D), jnp.float32),
        mesh=mesh,
        scratch_types=(
            pltpu.VMEM((RPW,), jnp.int32),
            pltpu.VMEM((NBUF, CHW, D), jnp.float32),
            pltpu.SemaphoreType.DMA((NBUF * GPW,)),
            pltpu.SemaphoreType.DMA((NBUF,)),
        ),
        name="sc_row_permute",
    )
    return run(perm, xf)


def kernel(X, perm):
    xf = X.reshape(ROWS, D)
    out = _permute_flat(perm, xf)
    return out.reshape(X.shape)
